# baseline (speedup 1.0000x reference)
import functools

import jax
import jax.numpy as jnp
from jax import lax
from jax.experimental import pallas as pl
from jax.experimental.pallas import tpu as pltpu, tpu_sc as plsc

MAX_SEQ = 512
INF = -1000000000.0
LANES = 16
NUM_WORKERS = 32


def _sc_body(src_stop, tgt_stop, num_sentences,
             enc_hbm, dec_hbm, cross_hbm,
             enc_v, dec_v, cross_v, sem):
    wid = lax.axis_index("s") * 2 + lax.axis_index("c")
    rows_per_tile = MAX_SEQ // NUM_WORKERS
    row0 = wid * rows_per_tile

    lane = lax.iota(jnp.int32, LANES)
    zero = jnp.zeros((LANES,), jnp.float32)
    inf = jnp.full((LANES,), INF, jnp.float32)

    for r in range(rows_per_tile):
        row = row0 + r
        t_enc = jnp.where(row < src_stop, src_stop, 0)
        t_cross = jnp.where(row < tgt_stop, src_stop, 0)
        t_dec = jnp.where(row < tgt_stop, row + 1, 0)
        for c in range(MAX_SEQ // LANES):
            col = c * LANES + lane
            cs = pl.ds(c * LANES, LANES)
            enc_v[r, cs] = jnp.where(col >= t_enc, inf, zero)
            dec_v[r, cs] = jnp.where(col >= t_dec, inf, zero)
            cross_v[r, cs] = jnp.where(col >= t_cross, inf, zero)

    handles = []
    for buf, out in ((enc_v, enc_hbm), (dec_v, dec_hbm), (cross_v, cross_hbm)):
        for b in range(num_sentences):
            handles.append(pltpu.async_copy(
                buf, out.at[b, pl.ds(row0, rows_per_tile), :], sem))
    for h in handles:
        h.wait()


def kernel(source_batch, target_batch):
    num_sentences = source_batch.shape[0]
    src_stop = source_batch.shape[1] + 1
    tgt_stop = target_batch.shape[1] + 1

    out_t = jax.ShapeDtypeStruct((num_sentences, MAX_SEQ, MAX_SEQ), jnp.float32)
    rows_per_tile = MAX_SEQ // NUM_WORKERS
    mesh = plsc.VectorSubcoreMesh(core_axis_name="c", subcore_axis_name="s")
    k = functools.partial(
        pl.kernel,
        mesh=mesh,
        out_type=(out_t, out_t, out_t),
        scratch_types=[
            pltpu.VMEM((rows_per_tile, MAX_SEQ), jnp.float32),
            pltpu.VMEM((rows_per_tile, MAX_SEQ), jnp.float32),
            pltpu.VMEM((rows_per_tile, MAX_SEQ), jnp.float32),
            pltpu.SemaphoreType.DMA,
        ],
    )(functools.partial(_sc_body, src_stop, tgt_stop, num_sentences))
    return k()


# speedup vs baseline: 1.3690x; 1.3690x over previous
import functools

import jax
import jax.numpy as jnp
from jax import lax
from jax.experimental import pallas as pl
from jax.experimental.pallas import tpu as pltpu, tpu_sc as plsc

MAX_SEQ = 512
INF = -1000000000.0
LANES = 16
NUM_WORKERS = 32


def _sc_body(tgt_stop, num_sentences, dec_hbm, dec_v, sem):
    wid = lax.axis_index("s") * 2 + lax.axis_index("c")
    rows_per_tile = MAX_SEQ // NUM_WORKERS
    row0 = wid * rows_per_tile

    lane = lax.iota(jnp.int32, LANES)
    zero = jnp.zeros((LANES,), jnp.float32)
    inf = jnp.full((LANES,), INF, jnp.float32)

    for r in range(rows_per_tile):
        row = row0 + r
        t_dec = jnp.where(row < tgt_stop, row + 1, 0)
        for c in range(MAX_SEQ // LANES):
            col = c * LANES + lane
            dec_v[r, pl.ds(c * LANES, LANES)] = jnp.where(col >= t_dec, inf, zero)

    handles = []
    for b in range(num_sentences):
        handles.append(pltpu.async_copy(
            dec_v, dec_hbm.at[b, pl.ds(row0, rows_per_tile), :], sem))
    for h in handles:
        h.wait()


def _tc_body(src_stop, tgt_stop, enc_ref, cross_ref):
    row = jax.lax.broadcasted_iota(jnp.int32, (MAX_SEQ, MAX_SEQ), 0)
    col = jax.lax.broadcasted_iota(jnp.int32, (MAX_SEQ, MAX_SEQ), 1)
    zero = jnp.zeros((MAX_SEQ, MAX_SEQ), jnp.float32)
    inf = jnp.full((MAX_SEQ, MAX_SEQ), INF, jnp.float32)
    enc_ref[0] = jnp.where((col >= src_stop) | (row >= src_stop), inf, zero)
    cross_ref[0] = jnp.where((col >= src_stop) | (row >= tgt_stop), inf, zero)


def kernel(source_batch, target_batch):
    num_sentences = source_batch.shape[0]
    src_stop = source_batch.shape[1] + 1
    tgt_stop = target_batch.shape[1] + 1

    out_t = jax.ShapeDtypeStruct((num_sentences, MAX_SEQ, MAX_SEQ), jnp.float32)

    rows_per_tile = MAX_SEQ // NUM_WORKERS
    mesh = plsc.VectorSubcoreMesh(core_axis_name="c", subcore_axis_name="s")
    dec_self = functools.partial(
        pl.kernel,
        mesh=mesh,
        out_type=out_t,
        scratch_types=[
            pltpu.VMEM((rows_per_tile, MAX_SEQ), jnp.float32),
            pltpu.SemaphoreType.DMA,
        ],
    )(functools.partial(_sc_body, tgt_stop, num_sentences))()

    spec = pl.BlockSpec((1, MAX_SEQ, MAX_SEQ), lambda i: (i, 0, 0))
    enc, cross = pl.pallas_call(
        functools.partial(_tc_body, src_stop, tgt_stop),
        grid=(num_sentences,),
        out_specs=(spec, spec),
        out_shape=(out_t, out_t),
    )()
    return enc, dec_self, cross


# speedup vs baseline: 2.7595x; 2.0157x over previous
import functools

import jax
import jax.numpy as jnp
from jax.experimental import pallas as pl

MAX_SEQ = 512
INF = -1000000000.0


def _mask_body(src_stop, tgt_stop, enc_ref, dec_self_ref, cross_ref):
    row = jax.lax.broadcasted_iota(jnp.int32, (MAX_SEQ, MAX_SEQ), 0)
    col = jax.lax.broadcasted_iota(jnp.int32, (MAX_SEQ, MAX_SEQ), 1)
    src_row = row >= src_stop
    src_col = col >= src_stop
    tgt_row = row >= tgt_stop
    tgt_col = col >= tgt_stop
    look_ahead = col > row
    zero = jnp.zeros((MAX_SEQ, MAX_SEQ), jnp.float32)
    inf = jnp.full((MAX_SEQ, MAX_SEQ), INF, jnp.float32)
    enc = jnp.where(src_row | src_col, inf, zero)
    dec = jnp.where(look_ahead | tgt_row | tgt_col, inf, zero)
    cross = jnp.where(src_col | tgt_row, inf, zero)
    for b in range(enc_ref.shape[0]):
        enc_ref[b] = enc
        dec_self_ref[b] = dec
        cross_ref[b] = cross


def kernel(source_batch, target_batch):
    num_sentences = source_batch.shape[0]
    src_stop = source_batch.shape[1] + 1
    tgt_stop = target_batch.shape[1] + 1

    out_shape = jax.ShapeDtypeStruct((num_sentences, MAX_SEQ, MAX_SEQ),
                                     jnp.float32)
    assert num_sentences % 2 == 0
    grid = (num_sentences // 2,)
    spec = pl.BlockSpec((2, MAX_SEQ, MAX_SEQ), lambda i: (i, 0, 0))
    enc, dec_self, cross = pl.pallas_call(
        functools.partial(_mask_body, src_stop, tgt_stop),
        grid=grid,
        out_specs=(spec, spec, spec),
        out_shape=(out_shape, out_shape, out_shape),
    )()
    return enc, dec_self, cross
